# baseline (speedup 1.0000x reference)
import functools

import jax
import jax.numpy as jnp
from jax import lax
from jax.experimental import pallas as pl
from jax.experimental.pallas import tpu as pltpu
from jax.experimental.pallas import tpu_sc as plsc

BN = 1024
BNQ = 512
VCHUNK = 576
RSUB = 32
_NC, _NS = 2, 16
_NW = _NC * _NS


def _vprep_body(vert_ref, sw_ref, tm_ref, vmat_ref, v2_ref, table_ref):
    v = vert_ref.shape[0]
    vp = vmat_ref.shape[0]
    j = sw_ref.shape[1]
    pv = jnp.concatenate(
        [vert_ref[...], jnp.full((vp - v, 3), 1e8, jnp.float32)], axis=0)
    vmat_ref[...] = jnp.concatenate(
        [-2.0 * pv, jnp.zeros((vp, 5), jnp.float32)], axis=1)
    vx, vy, vz = pv[:, 0:1], pv[:, 1:2], pv[:, 2:3]
    v2_ref[...] = vx * vx + vy * vy + vz * vz
    swp = jnp.concatenate(
        [sw_ref[...], jnp.zeros((vp - v, j), jnp.float32)], axis=0)
    table_ref[...] = lax.dot_general(
        swp, tm_ref[...], (((1,), (0,)), ((), ())),
        preferred_element_type=jnp.float32)


def _nn_body(cat_ref, vm_ref, v2_ref, idx_ref):
    xt = jnp.transpose(cat_ref[...])
    vp = vm_ref.shape[0]
    best = jnp.full((RSUB, BNQ), jnp.inf, jnp.float32)
    besti = jnp.zeros((RSUB, BNQ), jnp.int32)
    for c in range(vp // VCHUNK):
        m = lax.dot_general(
            vm_ref[c * VCHUNK:(c + 1) * VCHUNK, :], xt,
            (((1,), (0,)), ((), ())), preferred_element_type=jnp.float32)
        d = m + v2_ref[c * VCHUNK:(c + 1) * VCHUNK, :]
        for s in range(VCHUNK // RSUB):
            ch = d[s * RSUB:(s + 1) * RSUB, :]
            cid = c * (VCHUNK // RSUB) + s
            upd = ch < best
            best = jnp.minimum(best, ch)
            besti = jnp.where(upd, cid, besti)
    sio = lax.broadcasted_iota(jnp.int32, (RSUB, BNQ), 0)
    vv = besti * RSUB + sio
    gmin = jnp.min(best, axis=0, keepdims=True)
    vcand = jnp.where(best == gmin, vv, jnp.int32(2 ** 30))
    idx_ref[...] = jnp.clip(jnp.min(vcand, axis=0), 0, vp - 1)


def _make_sc_lbs(b_total):
    b_per_w = b_total // _NW
    nch = 2
    csz = b_per_w // nch
    groups = csz // 16
    mesh = plsc.VectorSubcoreMesh(core_axis_name="c", subcore_axis_name="s")

    @functools.partial(
        pl.kernel, mesh=mesh,
        out_type=[
            jax.ShapeDtypeStruct((b_total, 16), jnp.float32),
            jax.ShapeDtypeStruct((b_total, 4), jnp.float32),
            jax.ShapeDtypeStruct((b_total, 9), jnp.float32),
        ],
        compiler_params=pltpu.CompilerParams(
            use_tc_tiling_on_sc=False, needs_layout_passes=False),
        scratch_types=[
            pltpu.VMEM((csz,), jnp.int32),
            pltpu.VMEM((csz, 16), jnp.float32),
            pltpu.VMEM((csz, 8), jnp.float32),
            pltpu.VMEM((csz, 4), jnp.float32),
            pltpu.VMEM((csz, 9), jnp.float32),
            pltpu.SemaphoreType.DMA,
        ],
    )
    def fused(table_hbm, idx_hbm, cat_hbm, t_out, xb_out, rb_out,
              idx_v, t_v, cq_v, xb_v, rb_v, sem):
        wid = lax.axis_index("s") * _NC + lax.axis_index("c")

        iota = lax.broadcasted_iota(jnp.int32, (16,), 0)

        def col(k):
            return jnp.full((16,), k, jnp.int32)

        def body(g, carry):
            row = g * 16 + iota
            t = [plsc.load_gather(t_v, [row, col(k)]) for k in range(16)]
            px = plsc.load_gather(cq_v, [row, col(0)])
            py = plsc.load_gather(cq_v, [row, col(1)])
            pz = plsc.load_gather(cq_v, [row, col(2)])
            qr = plsc.load_gather(cq_v, [row, col(3)])
            qx = plsc.load_gather(cq_v, [row, col(4)])
            qy = plsc.load_gather(cq_v, [row, col(5)])
            qz = plsc.load_gather(cq_v, [row, col(6)])
            ss = qr * qr + qx * qx + qy * qy + qz * qz
            seed = jnp.int32(0x5F3759DF) - lax.shift_right_logical(
                plsc.bitcast(ss, jnp.int32), 1)
            y = plsc.bitcast(seed, jnp.float32)
            for _ in range(3):
                y = y * (1.5 - 0.5 * ss * y * y)
            r, xq, yq, zq = qr * y, qx * y, qy * y, qz * y
            rm = [
                1 - 2 * (yq * yq + zq * zq), 2 * (xq * yq - r * zq), 2 * (xq * zq + r * yq),
                2 * (xq * yq + r * zq), 1 - 2 * (xq * xq + zq * zq), 2 * (yq * zq - r * xq),
                2 * (xq * zq - r * yq), 2 * (yq * zq + r * xq), 1 - 2 * (xq * xq + yq * yq),
            ]
            for i in range(3):
                xb = t[4 * i] * px + t[4 * i + 1] * py + t[4 * i + 2] * pz + t[4 * i + 3]
                plsc.store_scatter(xb_v, [row, col(i)], xb)
                for j in range(3):
                    rb = (t[4 * i] * rm[j] + t[4 * i + 1] * rm[3 + j]
                          + t[4 * i + 2] * rm[6 + j])
                    plsc.store_scatter(rb_v, [row, col(3 * i + j)], rb)
            return carry

        for ch in range(nch):
            base = wid * b_per_w + ch * csz
            pltpu.sync_copy(idx_hbm.at[pl.ds(base, csz)], idx_v)
            pltpu.sync_copy(cat_hbm.at[pl.ds(base, csz)], cq_v)
            pltpu.async_copy(table_hbm.at[idx_v], t_v, sem).wait()
            lax.fori_loop(0, groups, body, 0)
            pltpu.sync_copy(t_v, t_out.at[pl.ds(base, csz)])
            pltpu.sync_copy(xb_v, xb_out.at[pl.ds(base, csz)])
            pltpu.sync_copy(rb_v, rb_out.at[pl.ds(base, csz)])

    return fused


def kernel(xyz, smpl_verts, skinning_weights, transforms_mat, rotation):
    n = xyz.shape[0]
    v = smpl_verts.shape[0]
    j = skinning_weights.shape[1]
    npad = -(-n // BN) * BN
    vp = -(-v // VCHUNK) * VCHUNK

    tm16 = transforms_mat.reshape(j, 16).astype(jnp.float32)

    vmat, v2, vt_table = pl.pallas_call(
        _vprep_body,
        out_shape=[
            jax.ShapeDtypeStruct((vp, 8), jnp.float32),
            jax.ShapeDtypeStruct((vp, 1), jnp.float32),
            jax.ShapeDtypeStruct((vp, 16), jnp.float32),
        ],
    )(smpl_verts, skinning_weights, tm16)

    catp = jnp.pad(jnp.concatenate([xyz, rotation], axis=1),
                   ((0, npad - n), (0, 1)))

    nbq = npad // BNQ
    idx3 = pl.pallas_call(
        _nn_body,
        grid=(nbq,),
        in_specs=[
            pl.BlockSpec((BNQ, 8), lambda i: (i, 0)),
            pl.BlockSpec((vp, 8), lambda i: (0, 0)),
            pl.BlockSpec((vp, 1), lambda i: (0, 0)),
        ],
        out_specs=pl.BlockSpec((BNQ,), lambda i: (i,)),
        out_shape=jax.ShapeDtypeStruct((npad,), jnp.int32),
    )(catp, vmat, v2)
    idx = idx3

    t16, xb4, rb9 = _make_sc_lbs(npad)(vt_table, idx, catp)

    x_bar = xb4[:n, :3]
    rotation_bar = rb9[:n].reshape(n, 3, 3)
    t_fwd = t16[:n].reshape(n, 4, 4)
    return x_bar, rotation_bar, t_fwd
